# baseline (speedup 1.0000x reference)
import functools

import jax
import jax.numpy as jnp
from jax import lax
from jax.experimental import pallas as pl
from jax.experimental.pallas import tpu as pltpu
from jax.experimental.pallas import tpu_sc as plsc

N = 5000
P = 5120
L = 16
NW = 32
GPW = P // L // NW
IOU_THRESH = 0.5

_GDN = lax.GatherDimensionNumbers(
    offset_dims=(), collapsed_slice_dims=(0,), start_index_map=(0,))


def _sweep_body(x1_h, y1_h, x2_h, y2_h, ar_h, cl_h, jst_h, keep_h, msk_h,
                keepo_h, hito_h,
                x1_t, y1_t, x2_t, y2_t, ar_t, cl_t, jst_t, keep_t, msk_t,
                tmp_t):
    core = lax.axis_index("c")
    sub = lax.axis_index("s")
    wid = sub * 2 + core
    iota16 = lax.iota(jnp.int32, L)

    pltpu.sync_copy(x1_h, x1_t)
    pltpu.sync_copy(y1_h, y1_t)
    pltpu.sync_copy(x2_h, x2_t)
    pltpu.sync_copy(y2_h, y2_t)
    pltpu.sync_copy(ar_h, ar_t)
    pltpu.sync_copy(cl_h, cl_t)
    pltpu.sync_copy(jst_h, jst_t)
    pltpu.sync_copy(keep_h, keep_t)
    pltpu.sync_copy(msk_h, msk_t)

    def vperm(vec, idx):
        return lax.gather(vec, idx[:, None], _GDN, (1,),
                          mode=lax.GatherScatterMode.PROMISE_IN_BOUNDS)

    def splat_at(ref, ga, lanev):
        return vperm(ref[pl.ds(ga, L)], lanev)

    def to_scalar(vec_f32):
        return (vec_f32 + iota16.astype(jnp.float32) * 0.0)[0]

    def any_lane(vf):
        for s in (1, 2, 4, 8):
            vf = jnp.maximum(vf, vperm(vf, jnp.bitwise_xor(iota16, s)))
        return vf

    def col_scan(i, js, active):
        spl = jnp.full((L,), i, jnp.int32)
        ga = (i // L) * L
        lanev = jnp.full((L,), i - ga, jnp.int32)
        x1i = splat_at(x1_t, ga, lanev)
        y1i = splat_at(y1_t, ga, lanev)
        x2i = splat_at(x2_t, ga, lanev)
        y2i = splat_at(y2_t, ga, lanev)
        ai = splat_at(ar_t, ga, lanev)
        ci = splat_at(cl_t, ga, lanev)
        g_hi = (i + L - 1) // L
        g_lo = jnp.where(active, js // L, g_hi)

        def jstep(g, accs):
            acck, acca = accs
            o = g * L
            x1j = x1_t[pl.ds(o, L)]
            y1j = y1_t[pl.ds(o, L)]
            x2j = x2_t[pl.ds(o, L)]
            y2j = y2_t[pl.ds(o, L)]
            aj = ar_t[pl.ds(o, L)]
            cj = cl_t[pl.ds(o, L)]
            kj = keep_t[pl.ds(o, L)]
            vj = o + iota16
            xx1 = jnp.maximum(x1j, x1i)
            yy1 = jnp.maximum(y1j, y1i)
            xx2 = jnp.minimum(x2j, x2i)
            yy2 = jnp.minimum(y2j, y2i)
            w = jnp.maximum(0.0, xx2 - xx1 + 1.0)
            h = jnp.maximum(0.0, yy2 - yy1 + 1.0)
            inter = w * h
            union = aj + ai - inter
            sup = jnp.logical_and(inter > IOU_THRESH * union, cj == ci)
            sup = jnp.logical_and(sup, vj < spl)
            supf = jnp.where(sup, 1.0, 0.0)
            kept = jnp.where(kj > 0.0, supf, 0.0)
            return (jnp.maximum(acck, kept), jnp.maximum(acca, supf))

        zz = jnp.zeros((L,), jnp.float32)
        acck, acca = lax.fori_loop(g_lo, g_hi, jstep, (zz, zz))
        return any_lane(acck), any_lane(acca)

    def group_step(k, _):
        ga = (k * NW + wid) * L

        def lane_step(lane, hitv):
            i = ga + lane
            lanev = jnp.full((L,), lane, jnp.int32)
            mk_spl = splat_at(msk_t, ga, lanev)
            flag_s = to_scalar(mk_spl)
            js = to_scalar(splat_at(jst_t, ga, lanev)).astype(jnp.int32)
            flag = flag_s > 0.0
            acck, acca = col_scan(i, js, flag)
            flagf = mk_spl
            updf = jnp.where(iota16 == lane, flagf, 0.0)
            kg = keep_t[pl.ds(ga, L)]
            keep_t[pl.ds(ga, L)] = kg * (1.0 - updf) + (1.0 - acck) * updf
            return hitv * (1.0 - updf) + acca * updf

        hitv = lax.fori_loop(0, L, lane_step,
                             jnp.zeros((L,), jnp.float32))
        pltpu.sync_copy(keep_t.at[pl.ds(ga, L)], keepo_h.at[pl.ds(ga, L)])
        tmp_t[...] = hitv
        pltpu.sync_copy(tmp_t, hito_h.at[pl.ds(ga, L)])
        return 0

    lax.fori_loop(0, GPW, group_step, 0)


def _make_sweep():
    mesh = plsc.VectorSubcoreMesh(core_axis_name="c", subcore_axis_name="s")
    f32 = jnp.float32
    return pl.kernel(
        _sweep_body,
        mesh=mesh,
        out_type=[jax.ShapeDtypeStruct((P,), f32),
                  jax.ShapeDtypeStruct((P,), f32)],
        scratch_types=[
            pltpu.VMEM((P,), f32), pltpu.VMEM((P,), f32),
            pltpu.VMEM((P,), f32), pltpu.VMEM((P,), f32),
            pltpu.VMEM((P,), f32), pltpu.VMEM((P,), f32),
            pltpu.VMEM((P,), f32), pltpu.VMEM((P,), f32),
            pltpu.VMEM((P,), f32),
            pltpu.VMEM((L,), f32),
        ],
    )


@jax.jit
def kernel(boxes, scores, idxs):
    max_coord = jnp.max(boxes)
    offsets = idxs.astype(boxes.dtype) * (max_coord + 1.0)
    b = boxes + offsets[:, None]
    x1, y1, x2, y2 = b[:, 0], b[:, 1], b[:, 2], b[:, 3]
    area = (x2 - x1 + 1.0) * (y2 - y1 + 1.0)

    ord1 = jnp.argsort(-scores)
    ord2 = jnp.argsort(idxs[ord1])
    order = ord1[ord2]

    pad = P - N
    pad_f = jnp.zeros((pad,), jnp.float32)

    def padded(v, pad_vals):
        return jnp.concatenate([v[order], pad_vals])

    x1s = padded(x1, pad_f)
    y1s = padded(y1, pad_f)
    x2s = padded(x2, pad_f)
    y2s = padded(y2, pad_f)
    areas = padded(area, jnp.ones((pad,), jnp.float32))
    clss = padded(idxs.astype(jnp.float32),
                  1000.0 + jnp.arange(pad, dtype=jnp.float32))

    counts = jnp.bincount(idxs, length=8)
    seg_start = jnp.concatenate(
        [jnp.zeros((1,), jnp.int32), jnp.cumsum(counts)[:-1].astype(jnp.int32)])
    cls_int = jnp.clip(clss.astype(jnp.int32), 0, 7)
    jstart = jnp.where(jnp.arange(P) < N, seg_start[cls_int],
                       jnp.arange(P)).astype(jnp.float32)

    sweep = _make_sweep()
    ones = jnp.ones((P,), jnp.float32)

    def cond(c):
        _, _, changed, _ = c
        return changed

    def body(c):
        keep, mask, _, it = c
        nk, hit = sweep(x1s, y1s, x2s, y2s, areas, clss, jstart, keep, mask)
        mask = jnp.where(it == 0, hit, mask)
        return nk, mask, jnp.any(nk != keep), it + 1

    keep_f, _, _, _ = lax.while_loop(
        cond, body, (ones, ones, jnp.bool_(True), jnp.int32(0)))

    keep_sorted = keep_f[:N] > 0.0
    scores_sorted = scores[order]
    out = jnp.zeros((N,), jnp.float32).at[order].set(
        jnp.where(keep_sorted, scores_sorted, 0.0))
    out = out.at[0].set(jnp.where(jnp.all(keep_sorted), out[0], 0.0))
    return out


# speedup vs baseline: 1.0373x; 1.0373x over previous
import functools

import jax
import jax.numpy as jnp
from jax import lax
from jax.experimental import pallas as pl
from jax.experimental.pallas import tpu as pltpu
from jax.experimental.pallas import tpu_sc as plsc

N = 5000
P = 5120
L = 16
NW = 32
GPW = P // L // NW
IOU_THRESH = 0.5

_GDN = lax.GatherDimensionNumbers(
    offset_dims=(), collapsed_slice_dims=(0,), start_index_map=(0,))


def _sweep_body(x1_h, y1_h, x2_h, y2_h, ar_h, cl_h, jst_h, keep_h, msk_h,
                keepo_h, hito_h,
                x1_t, y1_t, x2_t, y2_t, ar_t, cl_t, jst_t, keep_t, msk_t,
                keepn_t, tmp_t, sem):
    core = lax.axis_index("c")
    sub = lax.axis_index("s")
    wid = sub * 2 + core
    iota16 = lax.iota(jnp.int32, L)

    copies = [pltpu.async_copy(src, dst, sem) for src, dst in (
        (x1_h, x1_t), (y1_h, y1_t), (x2_h, x2_t), (y2_h, y2_t),
        (ar_h, ar_t), (cl_h, cl_t), (jst_h, jst_t), (keep_h, keep_t),
        (msk_h, msk_t))]
    for c in copies:
        c.wait()

    def vperm(vec, idx):
        return lax.gather(vec, idx[:, None], _GDN, (1,),
                          mode=lax.GatherScatterMode.PROMISE_IN_BOUNDS)

    def splat_at(ref, ga, lanev):
        return vperm(ref[pl.ds(ga, L)], lanev)

    def to_scalar(vec_f32):
        return (vec_f32 + iota16.astype(jnp.float32) * 0.0)[0]

    def any_lane(vf):
        for s in (1, 2, 4, 8):
            vf = jnp.maximum(vf, vperm(vf, jnp.bitwise_xor(iota16, s)))
        return vf

    def col_scan(i, js, active):
        spl = jnp.full((L,), i, jnp.int32)
        ga = (i // L) * L
        lanev = jnp.full((L,), i - ga, jnp.int32)
        x1i = splat_at(x1_t, ga, lanev)
        y1i = splat_at(y1_t, ga, lanev)
        x2i = splat_at(x2_t, ga, lanev)
        y2i = splat_at(y2_t, ga, lanev)
        ai = splat_at(ar_t, ga, lanev)
        ci = splat_at(cl_t, ga, lanev)
        g_hi = (i + L - 1) // L
        g_lo = jnp.where(active, js // L, g_hi)

        def jstep(g, accs):
            acck, acca = accs
            o = g * L
            x1j = x1_t[pl.ds(o, L)]
            y1j = y1_t[pl.ds(o, L)]
            x2j = x2_t[pl.ds(o, L)]
            y2j = y2_t[pl.ds(o, L)]
            aj = ar_t[pl.ds(o, L)]
            cj = cl_t[pl.ds(o, L)]
            kj = keep_t[pl.ds(o, L)]
            vj = o + iota16
            xx1 = jnp.maximum(x1j, x1i)
            yy1 = jnp.maximum(y1j, y1i)
            xx2 = jnp.minimum(x2j, x2i)
            yy2 = jnp.minimum(y2j, y2i)
            w = jnp.maximum(0.0, xx2 - xx1 + 1.0)
            h = jnp.maximum(0.0, yy2 - yy1 + 1.0)
            inter = w * h
            union = aj + ai - inter
            sup = jnp.logical_and(inter > IOU_THRESH * union, cj == ci)
            sup = jnp.logical_and(sup, vj < spl)
            supf = jnp.where(sup, 1.0, 0.0)
            kept = jnp.where(kj > 0.0, supf, 0.0)
            return (jnp.maximum(acck, kept), jnp.maximum(acca, supf))

        zz = jnp.zeros((L,), jnp.float32)
        acck, acca = lax.fori_loop(g_lo, g_hi, jstep, (zz, zz))
        return any_lane(acck), any_lane(acca)

    def group_step(k, _):
        ga = (k * NW + wid) * L
        keepn_t[pl.ds(ga, L)] = keep_t[pl.ds(ga, L)]

        def lane_step(lane, hitv):
            i = ga + lane
            lanev = jnp.full((L,), lane, jnp.int32)
            mk_spl = splat_at(msk_t, ga, lanev)
            flag_s = to_scalar(mk_spl)
            js = to_scalar(splat_at(jst_t, ga, lanev)).astype(jnp.int32)
            flag = flag_s > 0.0
            acck, acca = col_scan(i, js, flag)
            flagf = mk_spl
            updf = jnp.where(iota16 == lane, flagf, 0.0)
            kg = keepn_t[pl.ds(ga, L)]
            keepn_t[pl.ds(ga, L)] = kg * (1.0 - updf) + (1.0 - acck) * updf
            return hitv * (1.0 - updf) + acca * updf

        hitv = lax.fori_loop(0, L, lane_step,
                             jnp.zeros((L,), jnp.float32))
        pltpu.sync_copy(keepn_t.at[pl.ds(ga, L)], keepo_h.at[pl.ds(ga, L)])
        tmp_t[...] = hitv
        pltpu.sync_copy(tmp_t, hito_h.at[pl.ds(ga, L)])
        return 0

    lax.fori_loop(0, GPW, group_step, 0)


def _make_sweep():
    mesh = plsc.VectorSubcoreMesh(core_axis_name="c", subcore_axis_name="s")
    f32 = jnp.float32
    return pl.kernel(
        _sweep_body,
        mesh=mesh,
        out_type=[jax.ShapeDtypeStruct((P,), f32),
                  jax.ShapeDtypeStruct((P,), f32)],
        scratch_types=[
            pltpu.VMEM((P,), f32), pltpu.VMEM((P,), f32),
            pltpu.VMEM((P,), f32), pltpu.VMEM((P,), f32),
            pltpu.VMEM((P,), f32), pltpu.VMEM((P,), f32),
            pltpu.VMEM((P,), f32), pltpu.VMEM((P,), f32),
            pltpu.VMEM((P,), f32), pltpu.VMEM((P,), f32),
            pltpu.VMEM((L,), f32),
            pltpu.SemaphoreType.DMA,
        ],
    )


@jax.jit
def kernel(boxes, scores, idxs):
    max_coord = jnp.max(boxes)
    offsets = idxs.astype(boxes.dtype) * (max_coord + 1.0)
    b = boxes + offsets[:, None]
    x1, y1, x2, y2 = b[:, 0], b[:, 1], b[:, 2], b[:, 3]
    area = (x2 - x1 + 1.0) * (y2 - y1 + 1.0)

    ord1 = jnp.argsort(-scores)
    ord2 = jnp.argsort(idxs[ord1])
    order = ord1[ord2]

    pad = P - N
    pad_f = jnp.zeros((pad,), jnp.float32)

    def padded(v, pad_vals):
        return jnp.concatenate([v[order], pad_vals])

    x1s = padded(x1, pad_f)
    y1s = padded(y1, pad_f)
    x2s = padded(x2, pad_f)
    y2s = padded(y2, pad_f)
    areas = padded(area, jnp.ones((pad,), jnp.float32))
    clss = padded(idxs.astype(jnp.float32),
                  1000.0 + jnp.arange(pad, dtype=jnp.float32))

    counts = jnp.bincount(idxs, length=8)
    seg_start = jnp.concatenate(
        [jnp.zeros((1,), jnp.int32), jnp.cumsum(counts)[:-1].astype(jnp.int32)])
    cls_int = jnp.clip(clss.astype(jnp.int32), 0, 7)
    jstart = jnp.where(jnp.arange(P) < N, seg_start[cls_int],
                       jnp.arange(P)).astype(jnp.float32)

    sweep = _make_sweep()
    ones = jnp.ones((P,), jnp.float32)

    def cond(c):
        _, _, changed, _ = c
        return changed

    def body(c):
        keep, mask, _, it = c
        nk, hit = sweep(x1s, y1s, x2s, y2s, areas, clss, jstart, keep, mask)
        mask = jnp.where(it == 0, hit, mask)
        return nk, mask, jnp.any(nk != keep), it + 1

    keep_f, _, _, _ = lax.while_loop(
        cond, body, (ones, ones, jnp.bool_(True), jnp.int32(0)))

    keep_sorted = keep_f[:N] > 0.0
    scores_sorted = scores[order]
    out = jnp.zeros((N,), jnp.float32).at[order].set(
        jnp.where(keep_sorted, scores_sorted, 0.0))
    out = out.at[0].set(jnp.where(jnp.all(keep_sorted), out[0], 0.0))
    return out


# speedup vs baseline: 1.2086x; 1.1651x over previous
import functools

import jax
import jax.numpy as jnp
from jax import lax
from jax.experimental import pallas as pl
from jax.experimental.pallas import tpu as pltpu
from jax.experimental.pallas import tpu_sc as plsc

N = 5000
P = 5120
L = 16
NW = 32
GPW = P // L // NW
IOU_THRESH = 0.5

_GDN = lax.GatherDimensionNumbers(
    offset_dims=(), collapsed_slice_dims=(0,), start_index_map=(0,))


def _sweep_body(x1_h, y1_h, x2_h, y2_h, ar_h, cl_h, jst_h, keep_h, msk_h,
                keepo_h, hito_h,
                x1_t, y1_t, x2_t, y2_t, ar_t, cl_t, jst_t, keep_t, msk_t,
                keepn_t, tmp_t, sem):
    core = lax.axis_index("c")
    sub = lax.axis_index("s")
    wid = sub * 2 + core
    iota16 = lax.iota(jnp.int32, L)

    copies = [pltpu.async_copy(src, dst, sem) for src, dst in (
        (x1_h, x1_t), (y1_h, y1_t), (x2_h, x2_t), (y2_h, y2_t),
        (ar_h, ar_t), (cl_h, cl_t), (jst_h, jst_t), (keep_h, keep_t),
        (msk_h, msk_t))]
    for c in copies:
        c.wait()

    def vperm(vec, idx):
        return lax.gather(vec, idx[:, None], _GDN, (1,),
                          mode=lax.GatherScatterMode.PROMISE_IN_BOUNDS)

    def splat_at(ref, ga, lanev):
        return vperm(ref[pl.ds(ga, L)], lanev)

    def to_scalar(vec_f32):
        return (vec_f32 + iota16.astype(jnp.float32) * 0.0)[0]

    def any_lane(vf):
        for s in (1, 2, 4, 8):
            vf = jnp.maximum(vf, vperm(vf, jnp.bitwise_xor(iota16, s)))
        return vf

    def col_scan(i, js, active):
        spl = jnp.full((L,), i, jnp.int32)
        ga = (i // L) * L
        lanev = jnp.full((L,), i - ga, jnp.int32)
        x1i = splat_at(x1_t, ga, lanev)
        y1i = splat_at(y1_t, ga, lanev)
        x2i = splat_at(x2_t, ga, lanev)
        y2i = splat_at(y2_t, ga, lanev)
        ai = splat_at(ar_t, ga, lanev)
        ci = splat_at(cl_t, ga, lanev)
        g_hi = (i + L - 1) // L
        g_lo = jnp.where(active, js // L, g_hi)

        def jstep(g, accs):
            acck, acca = accs
            o = g * L
            x1j = x1_t[pl.ds(o, L)]
            y1j = y1_t[pl.ds(o, L)]
            x2j = x2_t[pl.ds(o, L)]
            y2j = y2_t[pl.ds(o, L)]
            aj = ar_t[pl.ds(o, L)]
            cj = cl_t[pl.ds(o, L)]
            kj = keep_t[pl.ds(o, L)]
            vj = o + iota16
            xx1 = jnp.maximum(x1j, x1i)
            yy1 = jnp.maximum(y1j, y1i)
            xx2 = jnp.minimum(x2j, x2i)
            yy2 = jnp.minimum(y2j, y2i)
            w = jnp.maximum(0.0, xx2 - xx1 + 1.0)
            h = jnp.maximum(0.0, yy2 - yy1 + 1.0)
            inter = w * h
            union = aj + ai - inter
            sup = jnp.logical_and(inter > IOU_THRESH * union, cj == ci)
            sup = jnp.logical_and(sup, vj < spl)
            supf = jnp.where(sup, 1.0, 0.0)
            kept = jnp.where(kj > 0.0, supf, 0.0)
            return (jnp.maximum(acck, kept), jnp.maximum(acca, supf))

        zz = jnp.zeros((L,), jnp.float32)
        acck, acca = lax.fori_loop(g_lo, g_hi, jstep, (zz, zz))
        return any_lane(acck), any_lane(acca)

    def group_step(k, _):
        ga = (k * NW + wid) * L
        keepn_t[pl.ds(ga, L)] = keep_t[pl.ds(ga, L)]

        def lane_step(lane, hitv):
            i = ga + lane
            lanev = jnp.full((L,), lane, jnp.int32)
            mk_spl = splat_at(msk_t, ga, lanev)
            flag_s = to_scalar(mk_spl)
            js = to_scalar(splat_at(jst_t, ga, lanev)).astype(jnp.int32)
            flag = flag_s > 0.0
            acck, acca = col_scan(i, js, flag)
            flagf = mk_spl
            updf = jnp.where(iota16 == lane, flagf, 0.0)
            kg = keepn_t[pl.ds(ga, L)]
            keepn_t[pl.ds(ga, L)] = kg * (1.0 - updf) + (1.0 - acck) * updf
            return hitv * (1.0 - updf) + acca * updf

        hitv = lax.fori_loop(0, L, lane_step,
                             jnp.zeros((L,), jnp.float32))
        pltpu.sync_copy(keepn_t.at[pl.ds(ga, L)], keepo_h.at[pl.ds(ga, L)])
        tmp_t[...] = hitv
        pltpu.sync_copy(tmp_t, hito_h.at[pl.ds(ga, L)])
        return 0

    lax.fori_loop(0, GPW, group_step, 0)


def _make_sweep():
    mesh = plsc.VectorSubcoreMesh(core_axis_name="c", subcore_axis_name="s")
    f32 = jnp.float32
    return pl.kernel(
        _sweep_body,
        mesh=mesh,
        out_type=[jax.ShapeDtypeStruct((P,), f32),
                  jax.ShapeDtypeStruct((P,), f32)],
        scratch_types=[
            pltpu.VMEM((P,), f32), pltpu.VMEM((P,), f32),
            pltpu.VMEM((P,), f32), pltpu.VMEM((P,), f32),
            pltpu.VMEM((P,), f32), pltpu.VMEM((P,), f32),
            pltpu.VMEM((P,), f32), pltpu.VMEM((P,), f32),
            pltpu.VMEM((P,), f32), pltpu.VMEM((P,), f32),
            pltpu.VMEM((L,), f32),
            pltpu.SemaphoreType.DMA,
        ],
    )


@jax.jit
def kernel(boxes, scores, idxs):
    max_coord = jnp.max(boxes)
    offsets = idxs.astype(boxes.dtype) * (max_coord + 1.0)
    b = boxes + offsets[:, None]

    ord1 = jnp.argsort(-scores)
    ord2 = jnp.argsort(idxs[ord1])
    order = ord1[ord2]

    pad = P - N
    bs = jnp.concatenate([b[order], jnp.zeros((pad, 4), jnp.float32)])
    x1s, y1s, x2s, y2s = bs[:, 0], bs[:, 1], bs[:, 2], bs[:, 3]
    areas = jnp.where(jnp.arange(P) < N,
                      (x2s - x1s + 1.0) * (y2s - y1s + 1.0), 1.0)
    clss = jnp.concatenate([idxs[order].astype(jnp.float32),
                            1000.0 + jnp.arange(pad, dtype=jnp.float32)])

    counts = jnp.bincount(idxs, length=8)
    seg_start = jnp.concatenate(
        [jnp.zeros((1,), jnp.int32), jnp.cumsum(counts)[:-1].astype(jnp.int32)])
    cls_int = jnp.clip(clss.astype(jnp.int32), 0, 7)
    jstart = jnp.where(jnp.arange(P) < N, seg_start[cls_int],
                       jnp.arange(P)).astype(jnp.float32)

    sweep = _make_sweep()
    ones = jnp.ones((P,), jnp.float32)

    def cond(c):
        _, _, changed, _ = c
        return changed

    def body(c):
        keep, mask, _, it = c
        nk, hit = sweep(x1s, y1s, x2s, y2s, areas, clss, jstart, keep, mask)
        mask = jnp.where(it == 0, hit, mask)
        return nk, mask, jnp.any(nk != keep), it + 1

    keep_f, _, _, _ = lax.while_loop(
        cond, body, (ones, ones, jnp.bool_(True), jnp.int32(0)))

    keep_sorted = keep_f[:N] > 0.0
    scores_sorted = scores[order]
    out = jnp.zeros((N,), jnp.float32).at[order].set(
        jnp.where(keep_sorted, scores_sorted, 0.0))
    out = out.at[0].set(jnp.where(jnp.all(keep_sorted), out[0], 0.0))
    return out
